# baseline (speedup 1.0000x reference)
import functools

import jax
import jax.numpy as jnp
from jax import lax
from jax.experimental import pallas as pl
from jax.experimental.pallas import tpu as pltpu
from jax.experimental.pallas import tpu_sc as plsc

_B, _L, _V, _D = 4096, 200, 1000000, 64
_NC, _NS = 2, 16
_NW = _NC * _NS
_PER_W = _B // _NW
_PB = 8
_NG = _D // 16
_C0 = 104
_C1 = _L - _C0


def _rsqrt(x):
    i = plsc.bitcast(x, jnp.int32)
    y = plsc.bitcast(jnp.int32(0x5F3759DF) - (i >> 1), jnp.float32)
    for _ in range(3):
        y = y * (1.5 - 0.5 * x * y * y)
    return y


@functools.partial(
    pl.kernel,
    out_type=jax.ShapeDtypeStruct((_B, _L, _D), jnp.float32),
    mesh=plsc.VectorSubcoreMesh(
        core_axis_name="c", subcore_axis_name="s", num_cores=_NC, num_subcores=_NS
    ),
    scratch_types=[
        pltpu.VMEM((_PB, _L), jnp.int32),
        pltpu.VMEM((_PB, _L), jnp.float32),
        pltpu.VMEM((_L, _D), jnp.float32),
        pltpu.VMEM((_L, _D), jnp.float32),
        pltpu.SemaphoreType.DMA,
    ],
    compiler_params=pltpu.CompilerParams(use_tc_tiling_on_sc=False, needs_layout_passes=False),
)
def _mz_embed(table_h, idx_h, int_h, out_h, idx_v, int_v, rows_v, out_v, sem):
    wid = lax.axis_index("s") * _NC + lax.axis_index("c")
    b0 = wid * _PER_W

    def chunk(ci, carry):
        bb = b0 + ci * _PB
        pltpu.sync_copy(idx_h.at[pl.ds(bb, _PB)], idx_v)
        pltpu.sync_copy(int_h.at[pl.ds(bb, _PB)], int_v)

        def one_b(pb, carry2):
            b = bb + pb
            cp1 = pltpu.async_copy(
                table_h.at[idx_v.at[pb, pl.ds(0, _C0)]],
                rows_v.at[pl.ds(0, _C0)], sem)
            cp2 = pltpu.async_copy(
                table_h.at[idx_v.at[pb, pl.ds(_C0, _C1)]],
                rows_v.at[pl.ds(_C0, _C1)], sem)
            cp1.wait()
            cp2.wait()

            def p1(li, accs):
                res = list(accs)
                for u in range(8):
                    l = li * 8 + u
                    for g in range(_NG):
                        v = rows_v[l, pl.ds(g * 16, 16)]
                        res[g] = res[g] + v * v
                return tuple(res)

            accs = lax.fori_loop(
                0, _L // 8, p1,
                tuple(jnp.zeros((16,), jnp.float32) for _ in range(_NG)))
            invs = tuple(_rsqrt(a) for a in accs)

            def p2(j, invs_c):
                base = jnp.minimum(j * 16, _L - 16)
                ivec = int_v[pb, pl.ds(base, 16)]
                for u in range(16):
                    s = ivec.at[jnp.full((16,), u, jnp.int32)].get(
                        mode="promise_in_bounds")
                    for g in range(_NG):
                        out_v[base + u, pl.ds(g * 16, 16)] = (
                            rows_v[base + u, pl.ds(g * 16, 16)]
                            * (s * invs_c[g]))
                return invs_c

            lax.fori_loop(0, (_L + 15) // 16, p2, invs)
            pltpu.sync_copy(out_v, out_h.at[b])
            return carry2

        lax.fori_loop(0, _PB, one_b, 0)
        return carry

    lax.fori_loop(0, _PER_W // _PB, chunk, 0)


def kernel(mz_idx, intensity, table):
    return _mz_embed(table, mz_idx.astype(jnp.int32), intensity)
